# baseline (speedup 1.0000x reference)
import functools

import jax
import jax.numpy as jnp
from jax import lax
from jax.experimental import pallas as pl
from jax.experimental.pallas import tpu as pltpu
from jax.experimental.pallas import tpu_sc as plsc

VOCAB = 100000
D = 64
NI = 4096
NJ = 50
B = NI * NJ

_info = plsc.get_sparse_core_info()
NC, NS = _info.num_cores, _info.num_subcores
NW = NC * NS
B_PER_W = B // NW
TASK = 128
TASKS_PER_W = B_PER_W // TASK
IB = NI // TASK
YS = 129

_mesh = plsc.VectorSubcoreMesh(core_axis_name="c", subcore_axis_name="s")


@functools.partial(
    pl.kernel,
    mesh=_mesh,
    out_type=jax.ShapeDtypeStruct((NJ, 8, IB, 8, TASK), jnp.float32),
    scratch_types=[
        pltpu.VMEM((B_PER_W,), jnp.int32),
        pltpu.VMEM((2 * B_PER_W,), jnp.int32),
        [pltpu.VMEM((2 * TASK, D // 2), jnp.float32) for _ in range(2)],
        [pltpu.VMEM((8, 8, YS), jnp.float32) for _ in range(2)],
        [pltpu.SemaphoreType.DMA for _ in range(2)],
        [pltpu.SemaphoreType.DMA for _ in range(2)],
    ],
    compiler_params=pltpu.CompilerParams(
        use_tc_tiling_on_sc=False, needs_layout_passes=False),
)
def _embed_kernel(idx_hbm, table_hbm, out_hbm, idx_v, pidx_v, gbuf, ybuf,
                  gsem, wsem):
    wid = lax.axis_index("s") * NC + lax.axis_index("c")
    base = wid * B_PER_W
    tg0 = wid * TASKS_PER_W

    pltpu.sync_copy(idx_hbm.at[pl.ds(base, B_PER_W)], idx_v)

    iota = lax.iota(jnp.int32, 16)
    i0s = [(iota + 16 * c) >> 3 for c in range(D // 16)]
    i1s = [(iota + 16 * c) & 7 for c in range(D // 16)]

    @plsc.parallel_loop(0, B_PER_W // 16, 1, unroll=4)
    def _(m):
        a = lax.shift_left(idx_v[pl.ds(m * 16, 16)], 2)
        pos = iota * 2 + m * 32
        plsc.store_scatter(pidx_v, [pos], a)
        plsc.store_scatter(pidx_v, [pos + 1], a + 1)

    def start_gather(u, b):
        pltpu.async_copy(
            table_hbm.at[pidx_v.at[pl.ds(u * 2 * TASK, TASK)]],
            gbuf[b].at[pl.ds(0, TASK)], gsem[b])
        pltpu.async_copy(
            table_hbm.at[pidx_v.at[pl.ds(u * 2 * TASK + TASK, TASK)]],
            gbuf[b].at[pl.ds(TASK, TASK)], gsem[b])

    def wait_gather(b):
        pltpu.make_async_copy(
            table_hbm.at[pl.ds(0, 2 * TASK)], gbuf[b], gsem[b]).wait()

    def start_write(u, b):
        tg = tg0 + u
        j = tg // IB
        it = tg % IB
        pltpu.async_copy(
            ybuf[b].at[:, :, pl.ds(0, TASK)],
            out_hbm.at[j, :, it], wsem[b])

    def wait_write(b):
        pltpu.make_async_copy(
            ybuf[b].at[:, :, pl.ds(0, TASK)], out_hbm.at[0, :, 0],
            wsem[b]).wait()

    def transpose(b):
        gb, yb = gbuf[b], ybuf[b]

        @plsc.parallel_loop(0, TASK, 1, unroll=8)
        def _(m):
            mv = jnp.full((16,), m, jnp.int32)
            for c in range(D // 16):
                x = gb[2 * m + c // 2, pl.ds((c % 2) * 16, 16)]
                plsc.store_scatter(yb, [i0s[c], i1s[c], mv], x)

    start_gather(0, 0)

    def step(u2, carry):
        u = 2 * u2
        start_gather(u + 1, 1)
        wait_gather(0)

        @pl.when(u2 > 0)
        def _():
            wait_write(0)

        transpose(0)
        start_write(u, 0)

        @pl.when(u2 < TASKS_PER_W // 2 - 1)
        def _():
            start_gather(u + 2, 0)

        wait_gather(1)

        @pl.when(u2 > 0)
        def _():
            wait_write(1)

        transpose(1)
        start_write(u + 1, 1)
        return carry

    lax.fori_loop(0, TASKS_PER_W // 2, step, 0)
    wait_write(0)
    wait_write(1)


_TW = 512
_TGRID = -(-VOCAB // _TW)


@functools.partial(
    pl.pallas_call,
    out_shape=jax.ShapeDtypeStruct((VOCAB, 2 * D), jnp.float32),
    grid=(_TGRID,),
    in_specs=[pl.BlockSpec((D, _TW), lambda i: (0, i))],
    out_specs=pl.BlockSpec((_TW, 2 * D), lambda i: (i, 0)),
)
def _pad_transpose(tt_ref, out_ref):
    x = tt_ref[...]
    out_ref[:, 0:D] = x.T
    out_ref[:, D:] = jnp.zeros((_TW, D), jnp.float32)


def kernel(token_ids, table):
    flat = token_ids.T.reshape(-1).astype(jnp.int32)
    tv = _pad_transpose(table.T).reshape(4 * VOCAB, D // 2)
    out5 = _embed_kernel(flat, tv)
    return out5.transpose(2, 4, 0, 1, 3).reshape(NI, NJ, D)


# speedup vs baseline: 1.5443x; 1.5443x over previous
import functools

import jax
import jax.numpy as jnp
from jax import lax
from jax.experimental import pallas as pl
from jax.experimental.pallas import tpu as pltpu
from jax.experimental.pallas import tpu_sc as plsc

VOCAB = 100000
D = 64
NI = 4096
NJ = 50
B = NI * NJ

_info = plsc.get_sparse_core_info()
NC, NS = _info.num_cores, _info.num_subcores
NW = NC * NS
B_PER_W = B // NW
TASK = 128
TASKS_PER_W = B_PER_W // TASK
IB = NI // TASK
YS = 129

_mesh = plsc.VectorSubcoreMesh(core_axis_name="c", subcore_axis_name="s")


@functools.partial(
    pl.kernel,
    mesh=_mesh,
    out_type=jax.ShapeDtypeStruct((NJ, 8, IB, 8, TASK), jnp.float32),
    scratch_types=[
        pltpu.VMEM((B_PER_W,), jnp.int32),
        pltpu.VMEM((2 * B_PER_W,), jnp.int32),
        [pltpu.VMEM((2 * TASK, D // 2), jnp.float32) for _ in range(2)],
        [pltpu.VMEM((8, 8, YS), jnp.float32) for _ in range(2)],
        [pltpu.SemaphoreType.DMA for _ in range(2)],
        [pltpu.SemaphoreType.DMA for _ in range(2)],
    ],
    compiler_params=pltpu.CompilerParams(
        use_tc_tiling_on_sc=False, needs_layout_passes=False),
)
def _embed_kernel(idx_hbm, table_hbm, out_hbm, idx_v, pidx_v, gbuf, ybuf,
                  gsem, wsem):
    wid = lax.axis_index("s") * NC + lax.axis_index("c")
    base = wid * B_PER_W
    tg0 = wid * TASKS_PER_W

    pltpu.sync_copy(idx_hbm.at[pl.ds(base, B_PER_W)], idx_v)

    iota = lax.iota(jnp.int32, 16)
    i0s = [(iota + 16 * c) >> 3 for c in range(D // 16)]
    i1s = [(iota + 16 * c) & 7 for c in range(D // 16)]

    @plsc.parallel_loop(0, B_PER_W // 16, 1, unroll=8)
    def _(m):
        a = lax.shift_left(idx_v[pl.ds(m * 16, 16)], 2)
        pos = iota * 2 + m * 32
        plsc.store_scatter(pidx_v, [pos], a)
        plsc.store_scatter(pidx_v, [pos + 1], a + 1)

    def start_gather(u, b):
        pltpu.async_copy(
            table_hbm.at[pidx_v.at[pl.ds(u * 2 * TASK, TASK)]],
            gbuf[b].at[pl.ds(0, TASK)], gsem[b])
        pltpu.async_copy(
            table_hbm.at[pidx_v.at[pl.ds(u * 2 * TASK + TASK, TASK)]],
            gbuf[b].at[pl.ds(TASK, TASK)], gsem[b])

    def wait_gather(b):
        pltpu.make_async_copy(
            table_hbm.at[pl.ds(0, 2 * TASK)], gbuf[b], gsem[b]).wait()

    def start_write(u, b):
        tg = tg0 + u
        j = tg // IB
        it = tg % IB
        pltpu.async_copy(
            ybuf[b].at[:, :, pl.ds(0, TASK)],
            out_hbm.at[j, :, it], wsem[b])

    def wait_write(b):
        pltpu.make_async_copy(
            ybuf[b].at[:, :, pl.ds(0, TASK)], out_hbm.at[0, :, 0],
            wsem[b]).wait()

    def transpose(b):
        gb, yb = gbuf[b], ybuf[b]

        @plsc.parallel_loop(0, TASK, 1, unroll=16)
        def _(m):
            mv = jnp.full((16,), m, jnp.int32)
            for c in range(D // 16):
                x = gb[2 * m + c // 2, pl.ds((c % 2) * 16, 16)]
                plsc.store_scatter(yb, [i0s[c], i1s[c], mv], x)

    start_gather(0, 0)

    def step(u2, carry):
        u = 2 * u2
        start_gather(u + 1, 1)
        wait_gather(0)

        @pl.when(u2 > 0)
        def _():
            wait_write(0)

        transpose(0)
        start_write(u, 0)

        @pl.when(u2 < TASKS_PER_W // 2 - 1)
        def _():
            start_gather(u + 2, 0)

        wait_gather(1)

        @pl.when(u2 > 0)
        def _():
            wait_write(1)

        transpose(1)
        start_write(u + 1, 1)
        return carry

    lax.fori_loop(0, TASKS_PER_W // 2, step, 0)
    wait_write(0)
    wait_write(1)


def kernel(token_ids, table):
    flat = token_ids.T.reshape(-1).astype(jnp.int32)
    tv = jnp.pad(table, ((0, 0), (0, D))).reshape(4 * VOCAB, D // 2)
    out5 = _embed_kernel(flat, tv)
    return out5.transpose(2, 4, 0, 1, 3).reshape(NI, NJ, D)


# speedup vs baseline: 1.5732x; 1.0187x over previous
import functools

import jax
import jax.numpy as jnp
from jax import lax
from jax.experimental import pallas as pl
from jax.experimental.pallas import tpu as pltpu
from jax.experimental.pallas import tpu_sc as plsc

VOCAB = 100000
D = 64
NI = 4096
NJ = 50
B = NI * NJ

_info = plsc.get_sparse_core_info()
NC, NS = _info.num_cores, _info.num_subcores
NW = NC * NS
B_PER_W = B // NW
TASK = 128
TASKS_PER_W = B_PER_W // TASK
IB = NI // TASK
YS = 129

_mesh = plsc.VectorSubcoreMesh(core_axis_name="c", subcore_axis_name="s")


@functools.partial(
    pl.kernel,
    mesh=_mesh,
    out_type=jax.ShapeDtypeStruct((NJ, 8, IB, 8, TASK), jnp.float32),
    scratch_types=[
        pltpu.VMEM((B_PER_W,), jnp.int32),
        pltpu.VMEM((2 * B_PER_W,), jnp.int32),
        [pltpu.VMEM((2 * TASK, D // 2), jnp.float32) for _ in range(2)],
        [pltpu.VMEM((8, 8, YS), jnp.float32) for _ in range(2)],
        [pltpu.SemaphoreType.DMA for _ in range(2)],
        [pltpu.SemaphoreType.DMA for _ in range(2)],
    ],
    compiler_params=pltpu.CompilerParams(
        use_tc_tiling_on_sc=False, needs_layout_passes=False),
)
def _embed_kernel(idx_hbm, table_hbm, out_hbm, idx_v, pidx_v, gbuf, ybuf,
                  gsem, wsem):
    wid = lax.axis_index("s") * NC + lax.axis_index("c")
    base = wid * B_PER_W
    tg0 = wid * TASKS_PER_W

    pltpu.sync_copy(idx_hbm.at[pl.ds(base, B_PER_W)], idx_v)

    iota = lax.iota(jnp.int32, 16)
    i0s = [(iota + 16 * c) >> 3 for c in range(D // 16)]
    i1s = [(iota + 16 * c) & 7 for c in range(D // 16)]

    @plsc.parallel_loop(0, B_PER_W // 16, 1, unroll=4)
    def _(m):
        a = lax.shift_left(idx_v[pl.ds(m * 16, 16)], 2)
        pos = iota * 2 + m * 32
        plsc.store_scatter(pidx_v, [pos], a)
        plsc.store_scatter(pidx_v, [pos + 1], a + 1)

    def start_gather(u, b):
        pltpu.async_copy(
            table_hbm.at[pidx_v.at[pl.ds(u * 2 * TASK, TASK)]],
            gbuf[b].at[pl.ds(0, TASK)], gsem[b])
        pltpu.async_copy(
            table_hbm.at[pidx_v.at[pl.ds(u * 2 * TASK + TASK, TASK)]],
            gbuf[b].at[pl.ds(TASK, TASK)], gsem[b])

    def wait_gather(b):
        pltpu.make_async_copy(
            table_hbm.at[pl.ds(0, 2 * TASK)], gbuf[b], gsem[b]).wait()

    def start_write(u, b):
        tg = tg0 + u
        j = tg // IB
        it = tg % IB
        pltpu.async_copy(
            ybuf[b].at[:, :, pl.ds(0, TASK)],
            out_hbm.at[j, :, it], wsem[b])

    def wait_write(b):
        pltpu.make_async_copy(
            ybuf[b].at[:, :, pl.ds(0, TASK)], out_hbm.at[0, :, 0],
            wsem[b]).wait()

    def transpose(b):
        gb, yb = gbuf[b], ybuf[b]

        @plsc.parallel_loop(0, TASK, 1, unroll=8)
        def _(m):
            mv = jnp.full((16,), m, jnp.int32)
            for c in range(D // 16):
                x = gb[2 * m + c // 2, pl.ds((c % 2) * 16, 16)]
                plsc.store_scatter(yb, [i0s[c], i1s[c], mv], x)

    start_gather(0, 0)

    def step(u2, carry):
        u = 2 * u2
        start_gather(u + 1, 1)
        wait_gather(0)

        @pl.when(u2 > 0)
        def _():
            wait_write(0)

        transpose(0)
        start_write(u, 0)

        @pl.when(u2 < TASKS_PER_W // 2 - 1)
        def _():
            start_gather(u + 2, 0)

        wait_gather(1)

        @pl.when(u2 > 0)
        def _():
            wait_write(1)

        transpose(1)
        start_write(u + 1, 1)
        return carry

    lax.fori_loop(0, TASKS_PER_W // 2, step, 0)
    wait_write(0)
    wait_write(1)


def kernel(token_ids, table):
    flat = token_ids.T.reshape(-1).astype(jnp.int32)
    tv = jnp.pad(table, ((0, 0), (0, D))).reshape(4 * VOCAB, D // 2)
    out5 = _embed_kernel(flat, tv)
    return out5.transpose(2, 4, 0, 1, 3).reshape(NI, NJ, D)
